# baseline (speedup 1.0000x reference)
import functools

import jax
import jax.numpy as jnp
from jax import lax
from jax.experimental import pallas as pl
from jax.experimental.pallas import tpu as pltpu
from jax.experimental.pallas import tpu_sc as plsc

B = 64
H = 512
W = 512
NC = 2
NS = 16
NW = NC * NS
SC_BASE = 32
ROWS = 32
CHUNKS_PER_SAMPLE = H // ROWS
CC = W // 16
LANES = 16
NEG = -1e30


@functools.partial(
    pl.kernel,
    out_type=jax.ShapeDtypeStruct((NW, LANES), jnp.float32),
    mesh=plsc.VectorSubcoreMesh(core_axis_name="c", subcore_axis_name="s"),
    scratch_types=[
        pltpu.VMEM((ROWS, W), jnp.float32),
        pltpu.VMEM((ROWS, W), jnp.float32),
        pltpu.VMEM((ROWS, W), jnp.int32),
        pltpu.VMEM((ROWS, W), jnp.int32),
        pltpu.VMEM((LANES,), jnp.int32),
        pltpu.VMEM((LANES,), jnp.float32),
        pltpu.SemaphoreType.DMA,
        pltpu.SemaphoreType.DMA,
        pltpu.SemaphoreType.DMA,
        pltpu.SemaphoreType.DMA,
    ],
)
def _sc_bag_reduce(x_hbm, z_hbm, catsb_hbm, out_hbm,
                   xb0, xb1, zb0, zb1, cat_v, res_v,
                   sx0, sx1, sz0, sz1):
    cid = lax.axis_index("c")
    sid = lax.axis_index("s")
    wid = sid * NC + cid
    smp = SC_BASE + wid

    xbufs = (xb0, xb1)
    zbufs = (zb0, zb1)
    sxs = (sx0, sx1)
    szs = (sz0, sz1)

    def start(k, par):
        pltpu.async_copy(x_hbm.at[smp, pl.ds(k * ROWS, ROWS), :],
                         xbufs[par], sxs[par])
        pltpu.async_copy(z_hbm.at[smp, pl.ds(k * ROWS, ROWS), :],
                         zbufs[par], szs[par])

    def wait(k, par):
        pltpu.make_async_copy(x_hbm.at[smp, pl.ds(k * ROWS, ROWS), :],
                              xbufs[par], sxs[par]).wait()
        pltpu.make_async_copy(z_hbm.at[smp, pl.ds(k * ROWS, ROWS), :],
                              zbufs[par], szs[par]).wait()

    pltpu.sync_copy(catsb_hbm.at[smp], cat_v)
    cat_vec = cat_v[...]

    start(0, 0)
    start(1, 1)

    def chunk_pair(g, vm):
        for par in range(2):
            k = 2 * g + par
            wait(k, par)
            xb = xbufs[par]
            zb = zbufs[par]

            def row_body(r, vmr, xb=xb, zb=zb):
                for c in range(CC):
                    z = zb[r, pl.ds(c * LANES, LANES)]
                    x = xb[r, pl.ds(c * LANES, LANES)]
                    vmr = jnp.maximum(vmr, jnp.where(z == cat_vec, x, NEG))
                return vmr

            vm = lax.fori_loop(0, ROWS, row_body, vm)

            @pl.when(k + 2 < CHUNKS_PER_SAMPLE)
            def _(k=k, par=par):
                start(k + 2, par)
        return vm

    vmax = lax.fori_loop(0, CHUNKS_PER_SAMPLE // 2, chunk_pair,
                         jnp.full((LANES,), NEG, dtype=jnp.float32))
    res_v[...] = vmax
    pltpu.sync_copy(res_v, out_hbm.at[wid])


def _tc_bag_body(cat_ref, x_ref, z_ref, out_ref):
    cat = cat_ref[0, 0, 0]
    x = x_ref[0]
    z = z_ref[0]
    out_ref[0, 0, 0] = jnp.max(jnp.where(z == cat, x, NEG))


def _loss_body(bagtc_ref, scv_ref, lab_ref, out_ref):
    bag_tc = bagtc_ref[...][:, 0, 0]
    bag_sc = jnp.max(scv_ref[...], axis=1)
    bag = jnp.concatenate([bag_tc, bag_sc])
    x = jnp.where(bag > -1e29, bag, 0.0)
    y = lab_ref[...]
    per = jnp.maximum(x, 0.0) - x * y + jnp.log1p(jnp.exp(-jnp.abs(x)))
    out_ref[0, 0] = jnp.sum(per) / B


def kernel(pixel_logits, zone_patches, cats, labels):
    x = pixel_logits.reshape(B, H, W)
    z = zone_patches
    cats_eff = jnp.where(cats > 0, cats, -1)
    cats_b = jnp.broadcast_to(cats_eff[:, None], (B, LANES))

    sc_res = _sc_bag_reduce(x, z, cats_b)

    bag_tc = pl.pallas_call(
        _tc_bag_body,
        grid=(SC_BASE,),
        in_specs=[
            pl.BlockSpec((1, 1, 1), lambda b: (b, 0, 0),
                         memory_space=pltpu.SMEM),
            pl.BlockSpec((1, H, W), lambda b: (b, 0, 0)),
            pl.BlockSpec((1, H, W), lambda b: (b, 0, 0)),
        ],
        out_specs=pl.BlockSpec((1, 1, 1), lambda b: (b, 0, 0),
                               memory_space=pltpu.SMEM),
        out_shape=jax.ShapeDtypeStruct((SC_BASE, 1, 1), jnp.float32),
    )(cats_eff[:SC_BASE, None, None], x[:SC_BASE], z[:SC_BASE])

    loss = pl.pallas_call(
        _loss_body,
        out_shape=jax.ShapeDtypeStruct((1, 1), jnp.float32),
        out_specs=pl.BlockSpec(memory_space=pltpu.SMEM),
    )(bag_tc, sc_res, labels)
    return loss[0, 0]


# speedup vs baseline: 1.8177x; 1.8177x over previous
import functools

import jax
import jax.numpy as jnp
from jax import lax
from jax.experimental import pallas as pl
from jax.experimental.pallas import tpu as pltpu
from jax.experimental.pallas import tpu_sc as plsc

B = 64
H = 512
W = 512
NC = 2
NS = 16
NW = NC * NS
SC_BASE = 32
ROWS = 32
CHUNKS_PER_SAMPLE = H // ROWS
CC = W // 16
LANES = 16
NEG = -1e30


@functools.partial(
    pl.kernel,
    out_type=jax.ShapeDtypeStruct((NW, LANES), jnp.float32),
    mesh=plsc.VectorSubcoreMesh(core_axis_name="c", subcore_axis_name="s"),
    scratch_types=[
        pltpu.VMEM((ROWS, W), jnp.float32),
        pltpu.VMEM((ROWS, W), jnp.float32),
        pltpu.VMEM((ROWS, W), jnp.int32),
        pltpu.VMEM((ROWS, W), jnp.int32),
        pltpu.VMEM((LANES,), jnp.int32),
        pltpu.VMEM((LANES,), jnp.float32),
        pltpu.SemaphoreType.DMA,
        pltpu.SemaphoreType.DMA,
        pltpu.SemaphoreType.DMA,
        pltpu.SemaphoreType.DMA,
    ],
)
def _sc_bag_reduce(x_hbm, z_hbm, catsb_hbm, out_hbm,
                   xb0, xb1, zb0, zb1, cat_v, res_v,
                   sx0, sx1, sz0, sz1):
    cid = lax.axis_index("c")
    sid = lax.axis_index("s")
    wid = sid * NC + cid
    smp = SC_BASE + wid

    xbufs = (xb0, xb1)
    zbufs = (zb0, zb1)
    sxs = (sx0, sx1)
    szs = (sz0, sz1)

    def start(k, par):
        pltpu.async_copy(x_hbm.at[smp, pl.ds(k * ROWS, ROWS), :],
                         xbufs[par], sxs[par])
        pltpu.async_copy(z_hbm.at[smp, pl.ds(k * ROWS, ROWS), :],
                         zbufs[par], szs[par])

    def wait(k, par):
        pltpu.make_async_copy(x_hbm.at[smp, pl.ds(k * ROWS, ROWS), :],
                              xbufs[par], sxs[par]).wait()
        pltpu.make_async_copy(z_hbm.at[smp, pl.ds(k * ROWS, ROWS), :],
                              zbufs[par], szs[par]).wait()

    pltpu.sync_copy(catsb_hbm.at[smp], cat_v)
    cat_vec = cat_v[...]

    start(0, 0)
    start(1, 1)

    def chunk_pair(g, vm):
        for par in range(2):
            k = 2 * g + par
            wait(k, par)
            xb = xbufs[par]
            zb = zbufs[par]

            def row_body(r, vmr, xb=xb, zb=zb):
                for c in range(CC):
                    z = zb[r, pl.ds(c * LANES, LANES)]
                    x = xb[r, pl.ds(c * LANES, LANES)]
                    vmr = jnp.maximum(vmr, jnp.where(z == cat_vec, x, NEG))
                return vmr

            vm = lax.fori_loop(0, ROWS, row_body, vm)

            @pl.when(k + 2 < CHUNKS_PER_SAMPLE)
            def _(k=k, par=par):
                start(k + 2, par)
        return vm

    vmax = lax.fori_loop(0, CHUNKS_PER_SAMPLE // 2, chunk_pair,
                         jnp.full((LANES,), NEG, dtype=jnp.float32))
    res_v[...] = vmax
    pltpu.sync_copy(res_v, out_hbm.at[wid])


def _tc_bag_body(cat_ref, x_ref, z_ref, out_ref):
    cat = cat_ref[0, 0, 0]
    x = x_ref[0]
    z = z_ref[0]
    out_ref[0, 0, 0] = jnp.max(jnp.where(z == cat, x, NEG))


def _loss_body(bagtc_ref, scv_ref, lab_ref, out_ref):
    bag_tc = bagtc_ref[...][:, 0, 0]
    bag_sc = jnp.max(scv_ref[...], axis=1)
    bag = jnp.concatenate([bag_tc, bag_sc])
    x = jnp.where(bag > -1e29, bag, 0.0)
    y = lab_ref[...]
    per = jnp.maximum(x, 0.0) - x * y + jnp.log1p(jnp.exp(-jnp.abs(x)))
    out_ref[0, 0] = jnp.sum(per) / B


def kernel(pixel_logits, zone_patches, cats, labels):
    x = pixel_logits.reshape(B, H, W)
    z = zone_patches
    cats_eff = jnp.where(cats > 0, cats, -1)
    cats_b = jnp.broadcast_to(cats_eff[:, None], (B, LANES))

    sc_res = _sc_bag_reduce(x, z, cats_b)

    bag_tc = pl.pallas_call(
        _tc_bag_body,
        grid=(SC_BASE,),
        in_specs=[
            pl.BlockSpec((1, 1, 1), lambda b: (b, 0, 0),
                         memory_space=pltpu.SMEM),
            pl.BlockSpec((1, H, W), lambda b: (b, 0, 0)),
            pl.BlockSpec((1, H, W), lambda b: (b, 0, 0)),
        ],
        out_specs=pl.BlockSpec((1, 1, 1), lambda b: (b, 0, 0),
                               memory_space=pltpu.SMEM),
        out_shape=jax.ShapeDtypeStruct((SC_BASE, 1, 1), jnp.float32),
    )(cats_eff[:, None, None], x, z)

    loss = pl.pallas_call(
        _loss_body,
        out_shape=jax.ShapeDtypeStruct((1, 1), jnp.float32),
        out_specs=pl.BlockSpec(memory_space=pltpu.SMEM),
    )(bag_tc, sc_res, labels)
    return loss[0, 0]
